# baseline (speedup 1.0000x reference)
import jax
import jax.numpy as jnp
from jax.experimental import pallas as pl
from jax.experimental.pallas import tpu as pltpu

H = 56
NVALID = H * H
MARGIN = 128
XLEN = MARGIN + NVALID + MARGIN


def _conv_body(x_ref, w_ref, m_ref, b_ref, o_ref, xq_ref):
    @pl.when(pl.program_id(0) == 0)
    def _():
        xq_ref[:, :MARGIN] = jnp.zeros((128, MARGIN), jnp.bfloat16)
        xq_ref[:, MARGIN + NVALID:] = jnp.zeros((128, MARGIN), jnp.bfloat16)

    c = x_ref.shape[1]
    xq_ref[:, MARGIN:MARGIN + NVALID] = (
        x_ref[0].astype(jnp.bfloat16).reshape(c, NVALID)
    )

    acc = None
    for p in range(5):
        halves = []
        for t in (2 * p, 2 * p + 1):
            if t < 9:
                kh, kw = t // 3, t % 3
                off = MARGIN - 57 + kw + kh * H
                xs = xq_ref[:, off:off + NVALID]
                if kw == 0:
                    xs = xs * m_ref[0]
                elif kw == 2:
                    xs = xs * m_ref[1]
            else:
                xs = jnp.zeros((128, NVALID), jnp.bfloat16)
            halves.append(xs)
        xcat = jnp.concatenate(halves, axis=0)
        d = jax.lax.dot_general(
            w_ref[p], xcat, (((1,), (0,)), ((), ())),
            preferred_element_type=jnp.float32,
        )
        acc = d if acc is None else acc + d
    o = o_ref.shape[1]
    o_ref[0] = (acc + b_ref[...]).reshape(o, H, H)


def kernel(x, W, b):
    n, c, h, w = x.shape
    o = W.shape[0]
    wr = jnp.transpose(W, (2, 3, 0, 1)).reshape(9, o, c)
    wr = jnp.concatenate([wr, jnp.zeros((1, o, c), wr.dtype)], axis=0)
    wp = wr.reshape(5, 2, o, c).transpose(0, 2, 1, 3).reshape(5, o, 2 * c)
    wp = wp.astype(jnp.bfloat16)
    b2 = b.reshape(o, 1)
    j = jnp.arange(NVALID)
    masks = jnp.stack([(j % H) != 0, (j % H) != (H - 1)])
    masks = jnp.broadcast_to(masks[:, None, :], (2, c, NVALID))
    masks = masks.astype(jnp.bfloat16)

    out = pl.pallas_call(
        _conv_body,
        out_shape=jax.ShapeDtypeStruct((n, o, H, H), jnp.float32),
        grid=(n,),
        in_specs=[
            pl.BlockSpec((1, c, H, H), lambda i: (i, 0, 0, 0)),
            pl.BlockSpec((5, o, 2 * c), lambda i: (0, 0, 0)),
            pl.BlockSpec((2, c, NVALID), lambda i: (0, 0, 0)),
            pl.BlockSpec((o, 1), lambda i: (0, 0)),
        ],
        out_specs=pl.BlockSpec((1, o, H, H), lambda i: (i, 0, 0, 0)),
        scratch_shapes=[pltpu.VMEM((c, XLEN), jnp.bfloat16)],
        compiler_params=pltpu.CompilerParams(
            dimension_semantics=("parallel",),
        ),
        name="conv3x3_flat",
    )(x, wp, masks, b2)

    return out


# speedup vs baseline: 3.7817x; 3.7817x over previous
import jax
import jax.numpy as jnp
from jax.experimental import pallas as pl
from jax.experimental.pallas import tpu as pltpu

H = 56
NPIX = H * H
MROWS = 64
XROWS = MROWS + NPIX + MROWS


def _conv_body(x_ref, w_ref, m_ref, b_ref, o_ref, xq_ref, xl_ref, xr_ref):
    @pl.when(pl.program_id(0) == 0)
    def _():
        xq_ref[:MROWS] = jnp.zeros((MROWS, 128), jnp.bfloat16)
        xq_ref[MROWS + NPIX:] = jnp.zeros((MROWS, 128), jnp.bfloat16)

    xq_ref[MROWS:MROWS + NPIX] = x_ref[0].astype(jnp.bfloat16)
    xl_ref[1:] = xq_ref[:XROWS - 1] * m_ref[0, 1:]
    xr_ref[:XROWS - 1] = xq_ref[1:] * m_ref[1, :XROWS - 1]

    parts = []
    for t in range(9):
        kh, kw = t // 3, t % 3
        base = 8 + kh * H
        src = (xl_ref, xq_ref, xr_ref)[kw]
        parts.append(src[base:base + NPIX])
    parts.append(jnp.zeros((NPIX, 128), jnp.bfloat16))
    xcat = jnp.concatenate(parts, axis=1)
    acc = jax.lax.dot_general(
        xcat, w_ref[...], (((1,), (0,)), ((), ())),
        preferred_element_type=jnp.float32,
    )
    o_ref[0] = acc + b_ref[...]


def kernel(x, W, b):
    n, c, h, w = x.shape
    o = W.shape[0]
    xt = jnp.transpose(x, (0, 2, 3, 1)).reshape(n, NPIX, c)
    wr = jnp.transpose(W, (2, 3, 1, 0)).reshape(9 * c, o)
    wr = jnp.concatenate([wr, jnp.zeros((c, o), wr.dtype)], axis=0)
    wr = wr.astype(jnp.bfloat16)
    b2 = b.reshape(1, o)
    r = jnp.arange(XROWS)
    masks = jnp.stack([(r - 8) % H != 0, (r - 8) % H != (H - 1)])
    masks = jnp.broadcast_to(masks[:, :, None], (2, XROWS, c))
    masks = masks.astype(jnp.bfloat16)

    out = pl.pallas_call(
        _conv_body,
        out_shape=jax.ShapeDtypeStruct((n, NPIX, o), jnp.float32),
        grid=(n,),
        in_specs=[
            pl.BlockSpec((1, NPIX, c), lambda i: (i, 0, 0)),
            pl.BlockSpec((10 * c, o), lambda i: (0, 0)),
            pl.BlockSpec((2, XROWS, c), lambda i: (0, 0, 0)),
            pl.BlockSpec((1, o), lambda i: (0, 0)),
        ],
        out_specs=pl.BlockSpec((1, NPIX, o), lambda i: (i, 0, 0)),
        scratch_shapes=[
            pltpu.VMEM((XROWS, c), jnp.bfloat16),
            pltpu.VMEM((XROWS, c), jnp.bfloat16),
            pltpu.VMEM((XROWS, c), jnp.bfloat16),
        ],
        compiler_params=pltpu.CompilerParams(
            dimension_semantics=("parallel",),
        ),
        name="conv3x3_nhwc",
    )(xt, wr, masks, b2)

    return out.reshape(n, h, w, o).transpose(0, 3, 1, 2)


# speedup vs baseline: 3.9933x; 1.0559x over previous
import jax
import jax.numpy as jnp
from jax.experimental import pallas as pl
from jax.experimental.pallas import tpu as pltpu

H = 56
NPIX = H * H
MROWS = 64
XROWS = MROWS + NPIX + MROWS


def _conv_body(x_ref, w_ref, m_ref, b_ref, o_ref, xq_ref, xl_ref, xr_ref):
    @pl.when(pl.program_id(0) == 0)
    def _():
        for im in range(2):
            xq_ref[im, :MROWS] = jnp.zeros((MROWS, 128), jnp.bfloat16)
            xq_ref[im, MROWS + NPIX:] = jnp.zeros((MROWS, 128), jnp.bfloat16)

    for im in range(2):
        xq = xq_ref.at[im]
        xl = xl_ref.at[im]
        xr = xr_ref.at[im]
        xq[MROWS:MROWS + NPIX] = x_ref[im].astype(jnp.bfloat16)
        xl[1:] = xq[:XROWS - 1] * m_ref[0, 1:]
        xr[:XROWS - 1] = xq[1:] * m_ref[1, :XROWS - 1]

        parts = []
        for t in range(9):
            kh, kw = t // 3, t % 3
            base = 8 + kh * H
            src = (xl, xq, xr)[kw]
            parts.append(src[base:base + NPIX])
        parts.append(jnp.zeros((NPIX, 128), jnp.bfloat16))
        xcat = jnp.concatenate(parts, axis=1)
        acc = jax.lax.dot_general(
            xcat, w_ref[...], (((1,), (0,)), ((), ())),
            preferred_element_type=jnp.float32,
        )
        o_ref[im] = acc + b_ref[...]


def kernel(x, W, b):
    n, c, h, w = x.shape
    o = W.shape[0]
    xt = jnp.transpose(x, (0, 2, 3, 1)).reshape(n, NPIX, c)
    wr = jnp.transpose(W, (2, 3, 1, 0)).reshape(9 * c, o)
    wr = jnp.concatenate([wr, jnp.zeros((c, o), wr.dtype)], axis=0)
    wr = wr.astype(jnp.bfloat16)
    b2 = b.reshape(1, o)
    r = jnp.arange(XROWS)
    masks = jnp.stack([(r - 8) % H != 0, (r - 8) % H != (H - 1)])
    masks = jnp.broadcast_to(masks[:, :, None], (2, XROWS, c))
    masks = masks.astype(jnp.bfloat16)

    out = pl.pallas_call(
        _conv_body,
        out_shape=jax.ShapeDtypeStruct((n, NPIX, o), jnp.float32),
        grid=(n // 2,),
        in_specs=[
            pl.BlockSpec((2, NPIX, c), lambda i: (i, 0, 0)),
            pl.BlockSpec((10 * c, o), lambda i: (0, 0)),
            pl.BlockSpec((2, XROWS, c), lambda i: (0, 0, 0)),
            pl.BlockSpec((1, o), lambda i: (0, 0)),
        ],
        out_specs=pl.BlockSpec((2, NPIX, o), lambda i: (i, 0, 0)),
        scratch_shapes=[
            pltpu.VMEM((2, XROWS, c), jnp.bfloat16),
            pltpu.VMEM((2, XROWS, c), jnp.bfloat16),
            pltpu.VMEM((2, XROWS, c), jnp.bfloat16),
        ],
        compiler_params=pltpu.CompilerParams(
            dimension_semantics=("parallel",),
        ),
        name="conv3x3_nhwc",
    )(xt, wr, masks, b2)

    return out.reshape(n, h, w, o).transpose(0, 3, 1, 2)


# speedup vs baseline: 3.9966x; 1.0008x over previous
import jax
import jax.numpy as jnp
from jax.experimental import pallas as pl
from jax.experimental.pallas import tpu as pltpu

H = 56
NPIX = H * H
MROWS = 64
XROWS = MROWS + NPIX + MROWS


def _conv_body(x_ref, w_ref, m_ref, b_ref, o_ref, xq_ref, xl_ref, xr_ref):
    @pl.when(pl.program_id(0) == 0)
    def _():
        for im in range(2):
            xq_ref[im, :MROWS] = jnp.zeros((MROWS, 128), jnp.bfloat16)
            xq_ref[im, MROWS + NPIX:] = jnp.zeros((MROWS, 128), jnp.bfloat16)

    for im in range(2):
        xq = xq_ref.at[im]
        xl = xl_ref.at[im]
        xr = xr_ref.at[im]
        xq[MROWS:MROWS + NPIX] = x_ref[im].astype(jnp.bfloat16)
        xl[1:] = xq[:XROWS - 1] * m_ref[0, 1:]
        xr[:XROWS - 1] = xq[1:] * m_ref[1, :XROWS - 1]

        parts = []
        for t in range(9):
            kh, kw = t // 3, t % 3
            base = 8 + kh * H
            src = (xl, xq, xr)[kw]
            parts.append(src[base:base + NPIX])
        parts.append(jnp.zeros((NPIX, 128), jnp.bfloat16))
        xcat = jnp.concatenate(parts, axis=1)
        acc = jax.lax.dot_general(
            xcat, w_ref[...], (((1,), (0,)), ((), ())),
            preferred_element_type=jnp.float32,
        )
        o_ref[im] = acc + b_ref[...]


def kernel(x, W, b):
    n, c, h, w = x.shape
    o = W.shape[0]
    xt = jnp.transpose(x, (0, 2, 3, 1)).reshape(n, NPIX, c)
    wr = jnp.transpose(W, (2, 3, 1, 0)).reshape(9 * c, o)
    wr = jnp.concatenate([wr, jnp.zeros((c, o), wr.dtype)], axis=0)
    wr = wr.astype(jnp.bfloat16)
    b2 = b.reshape(1, o)
    r = jnp.arange(XROWS)
    masks = jnp.stack([(r - 8) % H != 0, (r - 8) % H != (H - 1)])
    masks = jnp.broadcast_to(masks[:, :, None], (2, XROWS, c))
    masks = masks.astype(jnp.bfloat16)

    out = pl.pallas_call(
        _conv_body,
        out_shape=jax.ShapeDtypeStruct((n, NPIX, o), jnp.float32),
        grid=(n // 2,),
        in_specs=[
            pl.BlockSpec((2, NPIX, c), lambda i: (i, 0, 0)),
            pl.BlockSpec((10 * c, o), lambda i: (0, 0)),
            pl.BlockSpec((2, XROWS, c), lambda i: (0, 0, 0)),
            pl.BlockSpec((1, o), lambda i: (0, 0)),
        ],
        out_specs=pl.BlockSpec((2, NPIX, o), lambda i: (i, 0, 0)),
        scratch_shapes=[
            pltpu.VMEM((2, XROWS, c), jnp.bfloat16),
            pltpu.VMEM((2, XROWS, c), jnp.bfloat16),
            pltpu.VMEM((2, XROWS, c), jnp.bfloat16),
        ],
        compiler_params=pltpu.CompilerParams(
            dimension_semantics=("parallel",),
            vmem_limit_bytes=60 * 1024 * 1024,
        ),
        name="conv3x3_nhwc",
    )(xt, wr, masks, b2)

    return out.reshape(n, h, w, o).transpose(0, 3, 1, 2)


# speedup vs baseline: 3.9979x; 1.0003x over previous
import jax
import jax.numpy as jnp
from jax.experimental import pallas as pl
from jax.experimental.pallas import tpu as pltpu

H = 56
NPIX = H * H
MROWS = 64
XROWS = MROWS + NPIX + MROWS


def _conv_body(x_ref, w_ref, m_ref, b_ref, o_ref, xq_ref, xl_ref, xr_ref):
    @pl.when(pl.program_id(0) == 0)
    def _():
        for im in range(2):
            xq_ref[im, :MROWS] = jnp.zeros((MROWS, 128), jnp.bfloat16)
            xq_ref[im, MROWS + NPIX:] = jnp.zeros((MROWS, 128), jnp.bfloat16)

    for im in range(2):
        xq = xq_ref.at[im]
        xl = xl_ref.at[im]
        xr = xr_ref.at[im]
        xq[MROWS:MROWS + NPIX] = x_ref[im].astype(jnp.bfloat16)
        xl[1:] = xq[:XROWS - 1] * m_ref[0, 1:]
        xr[:XROWS - 1] = xq[1:] * m_ref[1, :XROWS - 1]

        parts = []
        for t in range(9):
            kh, kw = t // 3, t % 3
            base = 8 + kh * H
            src = (xl, xq, xr)[kw]
            parts.append(src[base:base + NPIX])
        xcat = jnp.concatenate(parts, axis=1)
        acc = jax.lax.dot_general(
            xcat, w_ref[...], (((1,), (0,)), ((), ())),
            preferred_element_type=jnp.float32,
        )
        o_ref[im] = acc + b_ref[...]


def kernel(x, W, b):
    n, c, h, w = x.shape
    o = W.shape[0]
    xt = jnp.transpose(x, (0, 2, 3, 1)).reshape(n, NPIX, c)
    wr = jnp.transpose(W, (2, 3, 1, 0)).reshape(9 * c, o)
    wr = wr.astype(jnp.bfloat16)
    b2 = b.reshape(1, o)
    r = jnp.arange(XROWS)
    masks = jnp.stack([(r - 8) % H != 0, (r - 8) % H != (H - 1)])
    masks = jnp.broadcast_to(masks[:, :, None], (2, XROWS, c))
    masks = masks.astype(jnp.bfloat16)

    out = pl.pallas_call(
        _conv_body,
        out_shape=jax.ShapeDtypeStruct((n, NPIX, o), jnp.float32),
        grid=(n // 2,),
        in_specs=[
            pl.BlockSpec((2, NPIX, c), lambda i: (i, 0, 0)),
            pl.BlockSpec((9 * c, o), lambda i: (0, 0)),
            pl.BlockSpec((2, XROWS, c), lambda i: (0, 0, 0)),
            pl.BlockSpec((1, o), lambda i: (0, 0)),
        ],
        out_specs=pl.BlockSpec((2, NPIX, o), lambda i: (i, 0, 0)),
        scratch_shapes=[
            pltpu.VMEM((2, XROWS, c), jnp.bfloat16),
            pltpu.VMEM((2, XROWS, c), jnp.bfloat16),
            pltpu.VMEM((2, XROWS, c), jnp.bfloat16),
        ],
        compiler_params=pltpu.CompilerParams(
            dimension_semantics=("parallel",),
            vmem_limit_bytes=60 * 1024 * 1024,
        ),
        name="conv3x3_nhwc",
    )(xt, wr, masks, b2)

    return out.reshape(n, h, w, o).transpose(0, 3, 1, 2)
